# baseline (device time: 126697 ns/iter reference)
import jax
import jax.numpy as jnp
from jax import lax
from jax.experimental import pallas as pl
from jax.experimental.pallas import tpu as pltpu

N_DEV = 16
B = 2
SQ = 128
D = 512
HQ_LOC = 8
DH = 64
GQA = 4


def kernel(x, Wq, Wo, K_ext, V_ext):
    idx = lax.axis_index("i")
    K_loc = lax.dynamic_slice_in_dim(K_ext, idx * (HQ_LOC // GQA), HQ_LOC // GQA, axis=2)
    V_loc = lax.dynamic_slice_in_dim(V_ext, idx * (HQ_LOC // GQA), HQ_LOC // GQA, axis=2)
    K_loc = jnp.transpose(K_loc, (0, 2, 1, 3))
    V_loc = jnp.transpose(V_loc, (0, 2, 1, 3))

    def body(x_ref, wq_ref, wo_ref, k_ref, v_ref, out_ref,
             comm_ref, send_sems, recv_sems, att_ref):
        my = lax.axis_index("i")
        left = lax.rem(my + N_DEV - 1, N_DEV)
        right = lax.rem(my + 1, N_DEV)

        barrier_sem = pltpu.get_barrier_semaphore()
        for nbr in (left, right):
            pl.semaphore_signal(
                barrier_sem, inc=1,
                device_id=(nbr,), device_id_type=pl.DeviceIdType.MESH,
            )
        pl.semaphore_wait(barrier_sem, 2)

        for b in range(B):
            qb = jnp.dot(x_ref[b], wq_ref[...],
                         preferred_element_type=jnp.float32)
            for h in range(HQ_LOC):
                c = h // GQA
                kb = k_ref[b, c]
                vb = v_ref[b, c]
                qh = qb[:, h * DH:(h + 1) * DH]
                s = lax.dot_general(
                    qh, kb, (((1,), (1,)), ((), ())),
                    preferred_element_type=jnp.float32,
                ) * 0.125
                m = jnp.max(s, axis=-1, keepdims=True)
                p = jnp.exp(s - m)
                l = jnp.sum(p, axis=-1, keepdims=True)
                o = jnp.dot(p, vb, preferred_element_type=jnp.float32) / l
                att_ref[b, :, h * DH:(h + 1) * DH] = o
            partial = jnp.dot(att_ref[b], wo_ref[...],
                              preferred_element_type=jnp.float32)
            out_ref[b] = partial
            comm_ref[0, b] = partial

        for h in range(N_DEV - 1):
            rdma = pltpu.make_async_remote_copy(
                src_ref=comm_ref.at[h],
                dst_ref=comm_ref.at[h + 1],
                send_sem=send_sems.at[h],
                recv_sem=recv_sems.at[h + 1],
                device_id=(right,),
                device_id_type=pl.DeviceIdType.MESH,
            )
            rdma.start()
            rdma.wait()
            out_ref[...] += comm_ref[h + 1]

    return pl.pallas_call(
        body,
        out_shape=jax.ShapeDtypeStruct((B, SQ, D), jnp.float32),
        in_specs=[pl.BlockSpec(memory_space=pltpu.VMEM)] * 5,
        out_specs=pl.BlockSpec(memory_space=pltpu.VMEM),
        scratch_shapes=[
            pltpu.VMEM((N_DEV, B, SQ, D), jnp.float32),
            pltpu.SemaphoreType.DMA((N_DEV,)),
            pltpu.SemaphoreType.DMA((N_DEV,)),
            pltpu.VMEM((B, SQ, D), jnp.float32),
        ],
        compiler_params=pltpu.CompilerParams(collective_id=0),
    )(x, Wq, Wo, K_loc, V_loc)


# device time: 42233 ns/iter; 3.0000x vs baseline; 3.0000x over previous
import jax
import jax.numpy as jnp
from jax import lax
from jax.experimental import pallas as pl
from jax.experimental.pallas import tpu as pltpu

N_DEV = 16
PLANE = 4
NZ = 4
B = 2
SQ = 128
D = 512
HQ_LOC = 8
DH = 64
GQA = 4
CROWS = SQ // PLANE


def kernel(x, Wq, Wo, K_ext, V_ext):
    idx = lax.axis_index("i")
    K_loc = lax.dynamic_slice_in_dim(K_ext, idx * (HQ_LOC // GQA), HQ_LOC // GQA, axis=2)
    V_loc = lax.dynamic_slice_in_dim(V_ext, idx * (HQ_LOC // GQA), HQ_LOC // GQA, axis=2)
    K_loc = jnp.transpose(K_loc, (0, 2, 1, 3))
    V_loc = jnp.transpose(V_loc, (0, 2, 1, 3))

    def body(x_ref, wq_ref, wo_ref, k_ref, v_ref, out_ref,
             pstore_ref, part_ref, rbufA, sstageA, zbuf, obuf, att_ref,
             sendA, recvA, sendB, recvB, sendC, recvC):
        my = lax.axis_index("i")
        z4 = (my // PLANE) * PLANE
        q = lax.rem(my, PLANE)
        pright = z4 + lax.rem(q + 1, PLANE)
        pleft = z4 + lax.rem(q + 3, PLANE)
        zright = lax.rem(my + PLANE, N_DEV)
        zleft = lax.rem(my + N_DEV - PLANE, N_DEV)

        barrier_sem = pltpu.get_barrier_semaphore()
        for nbr in (pleft, pright, zleft, zright):
            pl.semaphore_signal(
                barrier_sem, inc=1,
                device_id=(nbr,), device_id_type=pl.DeviceIdType.MESH,
            )
        pl.semaphore_wait(barrier_sem, 4)

        for b in range(B):
            qb = jnp.dot(x_ref[b], wq_ref[...],
                         preferred_element_type=jnp.float32)
            for h in range(HQ_LOC):
                c = h // GQA
                kb = k_ref[b, c]
                vb = v_ref[b, c]
                qh = qb[:, h * DH:(h + 1) * DH]
                s = lax.dot_general(
                    qh, kb, (((1,), (1,)), ((), ())),
                    preferred_element_type=jnp.float32,
                ) * 0.125
                m = jnp.max(s, axis=-1, keepdims=True)
                p = jnp.exp(s - m)
                l = jnp.sum(p, axis=-1, keepdims=True)
                o = jnp.dot(p, vb, preferred_element_type=jnp.float32) / l
                att_ref[b, :, h * DH:(h + 1) * DH] = o
            pstore_ref[b] = jnp.dot(att_ref[b], wo_ref[...],
                                    preferred_element_type=jnp.float32)

        for j in range(PLANE):
            off = lax.rem(q + j, PLANE) * CROWS
            for b in range(B):
                part_ref[j, b] = pstore_ref[b, pl.ds(off, CROWS), :]

        rdma = pltpu.make_async_remote_copy(
            src_ref=part_ref.at[0],
            dst_ref=rbufA.at[0],
            send_sem=sendA.at[0], recv_sem=recvA.at[0],
            device_id=(pright,), device_id_type=pl.DeviceIdType.MESH,
        )
        rdma.start()
        rdma.wait()
        for s in (1, 2):
            jloc = (-s) % PLANE
            sstageA[s - 1] = part_ref[jloc] + rbufA[s - 1]
            rdma = pltpu.make_async_remote_copy(
                src_ref=sstageA.at[s - 1],
                dst_ref=rbufA.at[s],
                send_sem=sendA.at[s], recv_sem=recvA.at[s],
                device_id=(pright,), device_id_type=pl.DeviceIdType.MESH,
            )
            rdma.start()
            rdma.wait()
        zbuf[0] = part_ref[1] + rbufA[2]

        for h in range(NZ - 1):
            rdma = pltpu.make_async_remote_copy(
                src_ref=zbuf.at[h],
                dst_ref=zbuf.at[h + 1],
                send_sem=sendB.at[h], recv_sem=recvB.at[h],
                device_id=(zright,), device_id_type=pl.DeviceIdType.MESH,
            )
            rdma.start()
            rdma.wait()
        obuf[1] = zbuf[0] + zbuf[1] + zbuf[2] + zbuf[3]

        for s in range(PLANE - 1):
            rdma = pltpu.make_async_remote_copy(
                src_ref=obuf.at[(1 - s) % PLANE],
                dst_ref=obuf.at[(-s) % PLANE],
                send_sem=sendC.at[s], recv_sem=recvC.at[s],
                device_id=(pright,), device_id_type=pl.DeviceIdType.MESH,
            )
            rdma.start()
            rdma.wait()

        for j in range(PLANE):
            off = lax.rem(q + j, PLANE) * CROWS
            for b in range(B):
                out_ref[b, pl.ds(off, CROWS), :] = obuf[j, b]

    chunk = (B, CROWS, D)
    return pl.pallas_call(
        body,
        out_shape=jax.ShapeDtypeStruct((B, SQ, D), jnp.float32),
        in_specs=[pl.BlockSpec(memory_space=pltpu.VMEM)] * 5,
        out_specs=pl.BlockSpec(memory_space=pltpu.VMEM),
        scratch_shapes=[
            pltpu.VMEM((B, SQ, D), jnp.float32),
            pltpu.VMEM((PLANE,) + chunk, jnp.float32),
            pltpu.VMEM((3,) + chunk, jnp.float32),
            pltpu.VMEM((2,) + chunk, jnp.float32),
            pltpu.VMEM((NZ,) + chunk, jnp.float32),
            pltpu.VMEM((PLANE,) + chunk, jnp.float32),
            pltpu.VMEM((B, SQ, D), jnp.float32),
            pltpu.SemaphoreType.DMA((3,)),
            pltpu.SemaphoreType.DMA((3,)),
            pltpu.SemaphoreType.DMA((3,)),
            pltpu.SemaphoreType.DMA((3,)),
            pltpu.SemaphoreType.DMA((3,)),
            pltpu.SemaphoreType.DMA((3,)),
        ],
        compiler_params=pltpu.CompilerParams(collective_id=0),
    )(x, Wq, Wo, K_loc, V_loc)


# device time: 40213 ns/iter; 3.1506x vs baseline; 1.0502x over previous
import jax
import jax.numpy as jnp
from jax import lax
from jax.experimental import pallas as pl
from jax.experimental.pallas import tpu as pltpu

N_DEV = 16
B = 2
SQ = 128
D = 512
HQ_LOC = 8
DH = 64
GQA = 4
R = B * SQ


def kernel(x, Wq, Wo, K_ext, V_ext):
    idx = lax.axis_index("i")
    K_loc = lax.dynamic_slice_in_dim(K_ext, idx * (HQ_LOC // GQA), HQ_LOC // GQA, axis=2)
    V_loc = lax.dynamic_slice_in_dim(V_ext, idx * (HQ_LOC // GQA), HQ_LOC // GQA, axis=2)
    K_loc = jnp.transpose(K_loc, (0, 2, 1, 3))
    V_loc = jnp.transpose(V_loc, (0, 2, 1, 3))

    def body(x_ref, wq_ref, wo_ref, k_ref, v_ref, out_ref,
             pstore, att_ref, w1, w2, w3, w4, r0, r1, r2, r3, g3, g2, g1,
             send_rs, recv_rs, send_ag, recv_ag):
        my = lax.axis_index("i")

        barrier_sem = pltpu.get_barrier_semaphore()
        for d in (1, 2, 4, 8):
            pl.semaphore_signal(
                barrier_sem, inc=1,
                device_id=(my ^ d,), device_id_type=pl.DeviceIdType.MESH,
            )
        pl.semaphore_wait(barrier_sem, 4)

        for b in range(B):
            qb = jnp.dot(x_ref[b], wq_ref[...],
                         preferred_element_type=jnp.float32)
            for h in range(HQ_LOC):
                c = h // GQA
                kb = k_ref[b, c]
                vb = v_ref[b, c]
                qh = qb[:, h * DH:(h + 1) * DH]
                s = lax.dot_general(
                    qh, kb, (((1,), (1,)), ((), ())),
                    preferred_element_type=jnp.float32,
                ) * 0.125
                m = jnp.max(s, axis=-1, keepdims=True)
                p = jnp.exp(s - m)
                l = jnp.sum(p, axis=-1, keepdims=True)
                o = jnp.dot(p, vb, preferred_element_type=jnp.float32) / l
                att_ref[b, :, h * DH:(h + 1) * DH] = o
            pstore[pl.ds(b * SQ, SQ), :] = jnp.dot(
                att_ref[b], wo_ref[...], preferred_element_type=jnp.float32)

        def exchange(src_lo, src_hi, dst_lo, dst_hi, bit0, d,
                     send_sem, recv_sem):
            @pl.when(bit0)
            def _():
                rd = pltpu.make_async_remote_copy(
                    src_ref=src_hi, dst_ref=dst_hi,
                    send_sem=send_sem, recv_sem=recv_sem,
                    device_id=(my ^ d,), device_id_type=pl.DeviceIdType.MESH,
                )
                rd.start()
                rd.wait()
            @pl.when(jnp.logical_not(bit0))
            def _():
                rd = pltpu.make_async_remote_copy(
                    src_ref=src_lo, dst_ref=dst_lo,
                    send_sem=send_sem, recv_sem=recv_sem,
                    device_id=(my ^ d,), device_id_type=pl.DeviceIdType.MESH,
                )
                rd.start()
                rd.wait()

        stages_in = [(pstore, r0, w1, R), (w1, r1, w2, R // 2),
                     (w2, r2, w3, R // 4), (w3, r3, w4, R // 8)]
        for k, (w_in, rst, w_out, S) in enumerate(stages_in):
            d = 1 << k
            bit0 = lax.rem(my // d, 2) == 0
            half = S // 2
            exchange(w_in.at[pl.ds(0, half)], w_in.at[pl.ds(half, half)],
                     rst, rst, bit0, d, send_rs.at[k], recv_rs.at[k])
            lo = w_in[pl.ds(0, half), :]
            hi = w_in[pl.ds(half, half), :]
            w_out[...] = jnp.where(bit0, lo, hi) + rst[...]

        stages_out = [(w4, g3, R // 16, 8), (g3, g2, R // 8, 4),
                      (g2, g1, R // 4, 2)]
        for j, (cur, gbuf, sh, d) in enumerate(stages_out):
            bit0 = lax.rem(my // d, 2) == 0
            @pl.when(bit0)
            def _(cur=cur, gbuf=gbuf, sh=sh):
                gbuf[pl.ds(0, sh), :] = cur[...]
            @pl.when(jnp.logical_not(bit0))
            def _(cur=cur, gbuf=gbuf, sh=sh):
                gbuf[pl.ds(sh, sh), :] = cur[...]
            exchange(cur, cur,
                     gbuf.at[pl.ds(0, sh)], gbuf.at[pl.ds(sh, sh)],
                     jnp.logical_not(bit0), d,
                     send_ag.at[j], recv_ag.at[j])

        bit0 = lax.rem(my, 2) == 0
        @pl.when(bit0)
        def _():
            out_ref[0] = g1[...]
        @pl.when(jnp.logical_not(bit0))
        def _():
            out_ref[1] = g1[...]
        exchange(g1, g1, out_ref.at[0], out_ref.at[1],
                 jnp.logical_not(bit0), 1, send_ag.at[3], recv_ag.at[3])

    return pl.pallas_call(
        body,
        out_shape=jax.ShapeDtypeStruct((B, SQ, D), jnp.float32),
        in_specs=[pl.BlockSpec(memory_space=pltpu.VMEM)] * 5,
        out_specs=pl.BlockSpec(memory_space=pltpu.VMEM),
        scratch_shapes=[
            pltpu.VMEM((R, D), jnp.float32),
            pltpu.VMEM((B, SQ, D), jnp.float32),
            pltpu.VMEM((R // 2, D), jnp.float32),
            pltpu.VMEM((R // 4, D), jnp.float32),
            pltpu.VMEM((R // 8, D), jnp.float32),
            pltpu.VMEM((R // 16, D), jnp.float32),
            pltpu.VMEM((R // 2, D), jnp.float32),
            pltpu.VMEM((R // 4, D), jnp.float32),
            pltpu.VMEM((R // 8, D), jnp.float32),
            pltpu.VMEM((R // 16, D), jnp.float32),
            pltpu.VMEM((R // 8, D), jnp.float32),
            pltpu.VMEM((R // 4, D), jnp.float32),
            pltpu.VMEM((R // 2, D), jnp.float32),
            pltpu.SemaphoreType.DMA((4,)),
            pltpu.SemaphoreType.DMA((4,)),
            pltpu.SemaphoreType.DMA((4,)),
            pltpu.SemaphoreType.DMA((4,)),
        ],
        compiler_params=pltpu.CompilerParams(collective_id=0),
    )(x, Wq, Wo, K_loc, V_loc)


# device time: 36706 ns/iter; 3.4517x vs baseline; 1.0955x over previous
import jax
import jax.numpy as jnp
from jax import lax
from jax.experimental import pallas as pl
from jax.experimental.pallas import tpu as pltpu

N_DEV = 16
B = 2
SQ = 128
D = 512
HQ_LOC = 8
DH = 64
GQA = 4
HKV = HQ_LOC // GQA
R = B * SQ
DC = D // 2


def kernel(x, Wq, Wo, K_ext, V_ext):
    idx = lax.axis_index("i")
    Hkv_tot = K_ext.shape[2]
    Kr = jnp.reshape(K_ext, (B, SQ, Hkv_tot * DH))
    Vr = jnp.reshape(V_ext, (B, SQ, Hkv_tot * DH))
    K_loc = lax.dynamic_slice_in_dim(Kr, idx * (HKV * DH), HKV * DH, axis=2)
    V_loc = lax.dynamic_slice_in_dim(Vr, idx * (HKV * DH), HKV * DH, axis=2)

    def body(x_ref, wq_ref, wo_ref, k_ref, v_ref, out_ref, att_ref,
             pstA, w1A, w2A, w3A, w4A, r0A, r1A, r2A, r3A, g3A, g2A, g1A,
             pstB, w1B, w2B, w3B, w4B, r0B, r1B, r2B, r3B, g3B, g2B, g1B,
             rs_sendA, rs_recvA, ag_sendA, ag_recvA,
             rs_sendB, rs_recvB, ag_sendB, ag_recvB):
        my = lax.axis_index("i")

        barrier_sem = pltpu.get_barrier_semaphore()
        for d in (1, 2, 4, 8):
            pl.semaphore_signal(
                barrier_sem, inc=1,
                device_id=(my ^ d,), device_id_type=pl.DeviceIdType.MESH,
            )
        pl.semaphore_wait(barrier_sem, 4)

        for b in range(B):
            qb = jnp.dot(x_ref[b], wq_ref[...],
                         preferred_element_type=jnp.float32)
            for h in range(HQ_LOC):
                c = h // GQA
                kb = k_ref[b, :, c * DH:(c + 1) * DH]
                vb = v_ref[b, :, c * DH:(c + 1) * DH]
                qh = qb[:, h * DH:(h + 1) * DH]
                s = lax.dot_general(
                    qh, kb, (((1,), (1,)), ((), ())),
                    preferred_element_type=jnp.float32,
                ) * 0.125
                m = jnp.max(s, axis=-1, keepdims=True)
                p = jnp.exp(s - m)
                l = jnp.sum(p, axis=-1, keepdims=True)
                o = jnp.dot(p, vb, preferred_element_type=jnp.float32) / l
                att_ref[b, :, h * DH:(h + 1) * DH] = o
            part = jnp.dot(att_ref[b], wo_ref[...],
                           preferred_element_type=jnp.float32)
            pstA[pl.ds(b * SQ, SQ), :] = part[:, :DC]
            pstB[pl.ds(b * SQ, SQ), :] = part[:, DC:]

        def bit_is0(d):
            return lax.rem(my // d, 2) == 0

        drains = []

        def rs_start(w_in, rst, S, d, send_sem, recv_sem):
            half = S // 2
            bit0 = bit_is0(d)
            @pl.when(bit0)
            def _():
                pltpu.make_async_remote_copy(
                    src_ref=w_in.at[pl.ds(half, half)], dst_ref=rst,
                    send_sem=send_sem, recv_sem=recv_sem,
                    device_id=(my ^ d,), device_id_type=pl.DeviceIdType.MESH,
                ).start()
            @pl.when(jnp.logical_not(bit0))
            def _():
                pltpu.make_async_remote_copy(
                    src_ref=w_in.at[pl.ds(0, half)], dst_ref=rst,
                    send_sem=send_sem, recv_sem=recv_sem,
                    device_id=(my ^ d,), device_id_type=pl.DeviceIdType.MESH,
                ).start()
            wd = pltpu.make_async_remote_copy(
                src_ref=rst, dst_ref=rst,
                send_sem=send_sem, recv_sem=recv_sem,
                device_id=(my ^ d,), device_id_type=pl.DeviceIdType.MESH,
            )
            return wd, bit0

        def rs_finish(wd, bit0, w_in, rst, w_out, S):
            half = S // 2
            wd.wait_recv()
            lo = w_in[pl.ds(0, half), :]
            hi = w_in[pl.ds(half, half), :]
            w_out[...] = jnp.where(bit0, lo, hi) + rst[...]
            drains.append(wd)

        RS_A = [(pstA, r0A, w1A, R, 1), (w1A, r1A, w2A, R // 2, 2),
                (w2A, r2A, w3A, R // 4, 4), (w3A, r3A, w4A, R // 8, 8)]
        RS_B = [(pstB, r0B, w1B, R, 4), (w1B, r1B, w2B, R // 2, 8),
                (w2B, r2B, w3B, R // 4, 1), (w3B, r3B, w4B, R // 8, 2)]
        for s in range(4):
            w_inA, rstA, w_outA, SA, dA = RS_A[s]
            w_inB, rstB, w_outB, SB, dB = RS_B[s]
            wdA, bitA = rs_start(w_inA, rstA, SA, dA,
                                 rs_sendA.at[s], rs_recvA.at[s])
            wdB, bitB = rs_start(w_inB, rstB, SB, dB,
                                 rs_sendB.at[s], rs_recvB.at[s])
            rs_finish(wdA, bitA, w_inA, rstA, w_outA, SA)
            rs_finish(wdB, bitB, w_inB, rstB, w_outB, SB)

        def ag_start(cur, gbuf, sh, d, send_sem, recv_sem):
            bit0 = bit_is0(d)
            @pl.when(bit0)
            def _():
                gbuf[pl.ds(0, sh), :] = cur[...]
                pltpu.make_async_remote_copy(
                    src_ref=cur, dst_ref=gbuf.at[pl.ds(0, sh)],
                    send_sem=send_sem, recv_sem=recv_sem,
                    device_id=(my ^ d,), device_id_type=pl.DeviceIdType.MESH,
                ).start()
            @pl.when(jnp.logical_not(bit0))
            def _():
                gbuf[pl.ds(sh, sh), :] = cur[...]
                pltpu.make_async_remote_copy(
                    src_ref=cur, dst_ref=gbuf.at[pl.ds(sh, sh)],
                    send_sem=send_sem, recv_sem=recv_sem,
                    device_id=(my ^ d,), device_id_type=pl.DeviceIdType.MESH,
                ).start()
            return pltpu.make_async_remote_copy(
                src_ref=cur, dst_ref=gbuf.at[pl.ds(0, sh)],
                send_sem=send_sem, recv_sem=recv_sem,
                device_id=(my ^ d,), device_id_type=pl.DeviceIdType.MESH,
            )

        def ag_final(cur, col0, d, send_sem, recv_sem):
            bit0 = bit_is0(d)
            @pl.when(bit0)
            def _():
                out_ref[0, :, col0:col0 + DC] = cur[...]
                pltpu.make_async_remote_copy(
                    src_ref=cur, dst_ref=out_ref.at[0, :, pl.ds(col0, DC)],
                    send_sem=send_sem, recv_sem=recv_sem,
                    device_id=(my ^ d,), device_id_type=pl.DeviceIdType.MESH,
                ).start()
            @pl.when(jnp.logical_not(bit0))
            def _():
                out_ref[1, :, col0:col0 + DC] = cur[...]
                pltpu.make_async_remote_copy(
                    src_ref=cur, dst_ref=out_ref.at[1, :, pl.ds(col0, DC)],
                    send_sem=send_sem, recv_sem=recv_sem,
                    device_id=(my ^ d,), device_id_type=pl.DeviceIdType.MESH,
                ).start()
            return pltpu.make_async_remote_copy(
                src_ref=cur, dst_ref=out_ref.at[0, :, pl.ds(col0, DC)],
                send_sem=send_sem, recv_sem=recv_sem,
                device_id=(my ^ d,), device_id_type=pl.DeviceIdType.MESH,
            )

        AG_A = [(w4A, g3A, R // 16, 8), (g3A, g2A, R // 8, 4),
                (g2A, g1A, R // 4, 2)]
        AG_B = [(w4B, g3B, R // 16, 2), (g3B, g2B, R // 8, 1),
                (g2B, g1B, R // 4, 8)]
        for j in range(3):
            curA, gbufA, shA, dA = AG_A[j]
            curB, gbufB, shB, dB = AG_B[j]
            wdA = ag_start(curA, gbufA, shA, dA, ag_sendA.at[j], ag_recvA.at[j])
            wdB = ag_start(curB, gbufB, shB, dB, ag_sendB.at[j], ag_recvB.at[j])
            wdA.wait_recv()
            wdB.wait_recv()
            drains.append(wdA)
            drains.append(wdB)

        wdA = ag_final(g1A, 0, 1, ag_sendA.at[3], ag_recvA.at[3])
        wdB = ag_final(g1B, DC, 4, ag_sendB.at[3], ag_recvB.at[3])
        wdA.wait_recv()
        wdB.wait_recv()
        drains.append(wdA)
        drains.append(wdB)

        for wd in drains:
            wd.wait_send()

    def stream_bufs():
        return [
            pltpu.VMEM((R, DC), jnp.float32),
            pltpu.VMEM((R // 2, DC), jnp.float32),
            pltpu.VMEM((R // 4, DC), jnp.float32),
            pltpu.VMEM((R // 8, DC), jnp.float32),
            pltpu.VMEM((R // 16, DC), jnp.float32),
            pltpu.VMEM((R // 2, DC), jnp.float32),
            pltpu.VMEM((R // 4, DC), jnp.float32),
            pltpu.VMEM((R // 8, DC), jnp.float32),
            pltpu.VMEM((R // 16, DC), jnp.float32),
            pltpu.VMEM((R // 8, DC), jnp.float32),
            pltpu.VMEM((R // 4, DC), jnp.float32),
            pltpu.VMEM((R // 2, DC), jnp.float32),
        ]

    return pl.pallas_call(
        body,
        out_shape=jax.ShapeDtypeStruct((B, SQ, D), jnp.float32),
        in_specs=[pl.BlockSpec(memory_space=pltpu.VMEM)] * 5,
        out_specs=pl.BlockSpec(memory_space=pltpu.VMEM),
        scratch_shapes=(
            [pltpu.VMEM((B, SQ, D), jnp.float32)]
            + stream_bufs() + stream_bufs()
            + [pltpu.SemaphoreType.DMA((4,))] * 8
        ),
        compiler_params=pltpu.CompilerParams(collective_id=0),
    )(x, Wq, Wo, K_loc, V_loc)


# device time: 34963 ns/iter; 3.6237x vs baseline; 1.0499x over previous
import jax
import jax.numpy as jnp
from jax import lax
from jax.experimental import pallas as pl
from jax.experimental.pallas import tpu as pltpu

N_DEV = 16
B = 2
SQ = 128
D = 512
HQ_LOC = 8
DH = 64
GQA = 4
HKV = HQ_LOC // GQA
R = B * SQ
DC = D // 2


def kernel(x, Wq, Wo, K_ext, V_ext):
    idx = lax.axis_index("i")
    K_loc = jnp.reshape(
        lax.dynamic_slice_in_dim(K_ext, idx * HKV, HKV, axis=2), (B, SQ, HKV * DH))
    V_loc = jnp.reshape(
        lax.dynamic_slice_in_dim(V_ext, idx * HKV, HKV, axis=2), (B, SQ, HKV * DH))

    def body(x_ref, wq_ref, wo_ref, k_ref, v_ref, out_ref, att_ref,
             pstA, w1A, w2A, w3A, w4A, r0A, r1A, r2A, r3A, g3A, g2A, g1A,
             pstB, w1B, w2B, w3B, w4B, r0B, r1B, r2B, r3B, g3B, g2B, g1B,
             rs_sendA, rs_recvA, ag_sendA, ag_recvA,
             rs_sendB, rs_recvB, ag_sendB, ag_recvB):
        my = lax.axis_index("i")

        barrier_sem = pltpu.get_barrier_semaphore()
        for d in (1, 2, 4, 8):
            pl.semaphore_signal(
                barrier_sem, inc=1,
                device_id=(my ^ d,), device_id_type=pl.DeviceIdType.MESH,
            )

        for b in range(B):
            qb = jnp.dot(x_ref[b], wq_ref[...],
                         preferred_element_type=jnp.float32)
            for h in range(HQ_LOC):
                c = h // GQA
                kb = k_ref[b, :, c * DH:(c + 1) * DH]
                vb = v_ref[b, :, c * DH:(c + 1) * DH]
                qh = qb[:, h * DH:(h + 1) * DH]
                s = lax.dot_general(
                    qh, kb, (((1,), (1,)), ((), ())),
                    preferred_element_type=jnp.float32,
                ) * 0.125
                m = jnp.max(s, axis=-1, keepdims=True)
                p = jnp.exp(s - m)
                l = jnp.sum(p, axis=-1, keepdims=True)
                o = jnp.dot(p, vb, preferred_element_type=jnp.float32) / l
                att_ref[b, :, h * DH:(h + 1) * DH] = o
            part = jnp.dot(att_ref[b], wo_ref[...],
                           preferred_element_type=jnp.float32)
            pstA[pl.ds(b * SQ, SQ), :] = part[:, :DC]
            pstB[pl.ds(b * SQ, SQ), :] = part[:, DC:]

        pl.semaphore_wait(barrier_sem, 4)

        def bit_is0(d):
            return lax.rem(my // d, 2) == 0

        drains = []

        def rs_start(w_in, rst, S, d, send_sem, recv_sem):
            half = S // 2
            bit0 = bit_is0(d)
            @pl.when(bit0)
            def _():
                pltpu.make_async_remote_copy(
                    src_ref=w_in.at[pl.ds(half, half)], dst_ref=rst,
                    send_sem=send_sem, recv_sem=recv_sem,
                    device_id=(my ^ d,), device_id_type=pl.DeviceIdType.MESH,
                ).start()
            @pl.when(jnp.logical_not(bit0))
            def _():
                pltpu.make_async_remote_copy(
                    src_ref=w_in.at[pl.ds(0, half)], dst_ref=rst,
                    send_sem=send_sem, recv_sem=recv_sem,
                    device_id=(my ^ d,), device_id_type=pl.DeviceIdType.MESH,
                ).start()
            wd = pltpu.make_async_remote_copy(
                src_ref=rst, dst_ref=rst,
                send_sem=send_sem, recv_sem=recv_sem,
                device_id=(my ^ d,), device_id_type=pl.DeviceIdType.MESH,
            )
            return wd, bit0

        def rs_finish(wd, bit0, w_in, rst, w_out, S):
            half = S // 2
            wd.wait_recv()
            lo = w_in[pl.ds(0, half), :]
            hi = w_in[pl.ds(half, half), :]
            w_out[...] = jnp.where(bit0, lo, hi) + rst[...]
            drains.append(wd)

        RS_A = [(pstA, r0A, w1A, R, 1), (w1A, r1A, w2A, R // 2, 2),
                (w2A, r2A, w3A, R // 4, 4), (w3A, r3A, w4A, R // 8, 8)]
        RS_B = [(pstB, r0B, w1B, R, 4), (w1B, r1B, w2B, R // 2, 8),
                (w2B, r2B, w3B, R // 4, 1), (w3B, r3B, w4B, R // 8, 2)]
        for s in range(4):
            w_inA, rstA, w_outA, SA, dA = RS_A[s]
            w_inB, rstB, w_outB, SB, dB = RS_B[s]
            wdA, bitA = rs_start(w_inA, rstA, SA, dA,
                                 rs_sendA.at[s], rs_recvA.at[s])
            wdB, bitB = rs_start(w_inB, rstB, SB, dB,
                                 rs_sendB.at[s], rs_recvB.at[s])
            rs_finish(wdA, bitA, w_inA, rstA, w_outA, SA)
            rs_finish(wdB, bitB, w_inB, rstB, w_outB, SB)

        def ag_start(cur, gbuf, sh, d, send_sem, recv_sem):
            bit0 = bit_is0(d)
            @pl.when(bit0)
            def _():
                gbuf[pl.ds(0, sh), :] = cur[...]
                pltpu.make_async_remote_copy(
                    src_ref=cur, dst_ref=gbuf.at[pl.ds(0, sh)],
                    send_sem=send_sem, recv_sem=recv_sem,
                    device_id=(my ^ d,), device_id_type=pl.DeviceIdType.MESH,
                ).start()
            @pl.when(jnp.logical_not(bit0))
            def _():
                gbuf[pl.ds(sh, sh), :] = cur[...]
                pltpu.make_async_remote_copy(
                    src_ref=cur, dst_ref=gbuf.at[pl.ds(sh, sh)],
                    send_sem=send_sem, recv_sem=recv_sem,
                    device_id=(my ^ d,), device_id_type=pl.DeviceIdType.MESH,
                ).start()
            return pltpu.make_async_remote_copy(
                src_ref=cur, dst_ref=gbuf.at[pl.ds(0, sh)],
                send_sem=send_sem, recv_sem=recv_sem,
                device_id=(my ^ d,), device_id_type=pl.DeviceIdType.MESH,
            )

        def ag_final(cur, col0, d, send_sem, recv_sem):
            bit0 = bit_is0(d)
            @pl.when(bit0)
            def _():
                out_ref[0, :, col0:col0 + DC] = cur[...]
                pltpu.make_async_remote_copy(
                    src_ref=cur, dst_ref=out_ref.at[0, :, pl.ds(col0, DC)],
                    send_sem=send_sem, recv_sem=recv_sem,
                    device_id=(my ^ d,), device_id_type=pl.DeviceIdType.MESH,
                ).start()
            @pl.when(jnp.logical_not(bit0))
            def _():
                out_ref[1, :, col0:col0 + DC] = cur[...]
                pltpu.make_async_remote_copy(
                    src_ref=cur, dst_ref=out_ref.at[1, :, pl.ds(col0, DC)],
                    send_sem=send_sem, recv_sem=recv_sem,
                    device_id=(my ^ d,), device_id_type=pl.DeviceIdType.MESH,
                ).start()
            return pltpu.make_async_remote_copy(
                src_ref=cur, dst_ref=out_ref.at[0, :, pl.ds(col0, DC)],
                send_sem=send_sem, recv_sem=recv_sem,
                device_id=(my ^ d,), device_id_type=pl.DeviceIdType.MESH,
            )

        AG_A = [(w4A, g3A, R // 16, 8), (g3A, g2A, R // 8, 4),
                (g2A, g1A, R // 4, 2)]
        AG_B = [(w4B, g3B, R // 16, 2), (g3B, g2B, R // 8, 1),
                (g2B, g1B, R // 4, 8)]
        for j in range(3):
            curA, gbufA, shA, dA = AG_A[j]
            curB, gbufB, shB, dB = AG_B[j]
            wdA = ag_start(curA, gbufA, shA, dA, ag_sendA.at[j], ag_recvA.at[j])
            wdB = ag_start(curB, gbufB, shB, dB, ag_sendB.at[j], ag_recvB.at[j])
            wdA.wait_recv()
            wdB.wait_recv()
            drains.append(wdA)
            drains.append(wdB)

        wdA = ag_final(g1A, 0, 1, ag_sendA.at[3], ag_recvA.at[3])
        wdB = ag_final(g1B, DC, 4, ag_sendB.at[3], ag_recvB.at[3])
        wdA.wait_recv()
        wdB.wait_recv()
        drains.append(wdA)
        drains.append(wdB)

        for wd in drains:
            wd.wait_send()

    def stream_bufs():
        return [
            pltpu.VMEM((R, DC), jnp.float32),
            pltpu.VMEM((R // 2, DC), jnp.float32),
            pltpu.VMEM((R // 4, DC), jnp.float32),
            pltpu.VMEM((R // 8, DC), jnp.float32),
            pltpu.VMEM((R // 16, DC), jnp.float32),
            pltpu.VMEM((R // 2, DC), jnp.float32),
            pltpu.VMEM((R // 4, DC), jnp.float32),
            pltpu.VMEM((R // 8, DC), jnp.float32),
            pltpu.VMEM((R // 16, DC), jnp.float32),
            pltpu.VMEM((R // 8, DC), jnp.float32),
            pltpu.VMEM((R // 4, DC), jnp.float32),
            pltpu.VMEM((R // 2, DC), jnp.float32),
        ]

    return pl.pallas_call(
        body,
        out_shape=jax.ShapeDtypeStruct((B, SQ, D), jnp.float32),
        in_specs=[pl.BlockSpec(memory_space=pltpu.VMEM)] * 5,
        out_specs=pl.BlockSpec(memory_space=pltpu.VMEM),
        scratch_shapes=(
            [pltpu.VMEM((B, SQ, D), jnp.float32)]
            + stream_bufs() + stream_bufs()
            + [pltpu.SemaphoreType.DMA((4,))] * 8
        ),
        compiler_params=pltpu.CompilerParams(collective_id=0),
    )(x, Wq, Wo, K_loc, V_loc)
